# baseline (speedup 1.0000x reference)
import functools

import jax
import jax.numpy as jnp
from jax import lax
from jax.experimental import pallas as pl
from jax.experimental.pallas import tpu as pltpu
from jax.experimental.pallas import tpu_sc as plsc

BATCH = 4
SEQ = 2048
HIDDEN = 2048
N_ROWS = BATCH * SEQ
NUM_CORES = 2
NUM_SUBCORES = 16
NUM_WORKERS = NUM_CORES * NUM_SUBCORES
ROWS_PER_W = N_ROWS // NUM_WORKERS
CHUNK = 4
NBUF = 4
NUM_CHUNKS = ROWS_PER_W // CHUNK
NUM_STEPS = NUM_CHUNKS // NBUF
LANES = 16
VECS_PER_ROW = HIDDEN // LANES

_mesh = plsc.VectorSubcoreMesh(core_axis_name="c", subcore_axis_name="s")


@functools.partial(
    pl.kernel,
    mesh=_mesh,
    out_type=jax.ShapeDtypeStruct((N_ROWS, HIDDEN), jnp.float32),
    scratch_types=[
        pltpu.VMEM((NUM_CHUNKS, CHUNK), jnp.int32),
        pltpu.VMEM((NUM_CHUNKS, CHUNK), jnp.int32),
        [pltpu.VMEM((CHUNK, HIDDEN), jnp.float32) for _ in range(NBUF)],
        [pltpu.VMEM((CHUNK, HIDDEN), jnp.float32) for _ in range(NBUF)],
        [pltpu.SemaphoreType.DMA for _ in range(NBUF)],
        [pltpu.SemaphoreType.DMA for _ in range(NBUF)],
    ],
)
def _emb_kernel(idw_hbm, idp_hbm, wt_hbm, pt_hbm, out_hbm,
                idw_v, idp_v, wbufs, pbufs, gsems, ssems):
    wid = lax.axis_index("s") * NUM_CORES + lax.axis_index("c")
    base = pl.multiple_of(wid * ROWS_PER_W, ROWS_PER_W)
    pltpu.sync_copy(idw_hbm.at[wid], idw_v)
    pltpu.sync_copy(idp_hbm.at[wid], idp_v)

    def issue_gathers(ci, j):
        pltpu.async_copy(wt_hbm.at[idw_v.at[ci]], wbufs[j], gsems[j])
        pltpu.async_copy(pt_hbm.at[idp_v.at[ci]], pbufs[j], gsems[j])

    def wait_gathers(j):
        pltpu.make_async_copy(wt_hbm.at[idw_v.at[0]], wbufs[j], gsems[j]).wait()
        pltpu.make_async_copy(pt_hbm.at[idp_v.at[0]], pbufs[j], gsems[j]).wait()

    def wait_scatter(j):
        pltpu.make_async_copy(
            wbufs[j], out_hbm.at[pl.ds(base, CHUNK)], ssems[j]).wait()

    for j in range(NBUF):
        issue_gathers(j, j)

    def step(pi, _):
        for j in range(NBUF):
            ci = pi * NBUF + j
            wait_gathers(j)
            wbuf, pbuf = wbufs[j], pbufs[j]

            def vec_body(vi, _):
                col = pl.ds(vi * LANES, LANES)
                for r in range(CHUNK):
                    plsc.addupdate(wbuf.at[r, col], pbuf[r, col])
                return 0

            lax.fori_loop(0, VECS_PER_ROW, vec_body, 0)
            pltpu.async_copy(
                wbuf, out_hbm.at[pl.ds(base + ci * CHUNK, CHUNK)], ssems[j])

            nxt = ci + NBUF

            @pl.when(nxt < NUM_CHUNKS)
            def _():
                wait_scatter(j)
                issue_gathers(nxt, j)

        return 0

    lax.fori_loop(0, NUM_STEPS, step, 0)
    for j in range(NBUF):
        wait_scatter(j)


def kernel(input_ids, position_ids, word_table, pos_table):
    idw = jnp.transpose(input_ids).reshape(NUM_WORKERS, NUM_CHUNKS, CHUNK)
    idp = jnp.transpose(position_ids).reshape(NUM_WORKERS, NUM_CHUNKS, CHUNK)
    out = _emb_kernel(idw.astype(jnp.int32), idp.astype(jnp.int32),
                      word_table, pos_table)
    return out.reshape(SEQ, BATCH, HIDDEN)


# speedup vs baseline: 1.4346x; 1.4346x over previous
import functools

import jax
import jax.numpy as jnp
from jax import lax
from jax.experimental import pallas as pl
from jax.experimental.pallas import tpu as pltpu
from jax.experimental.pallas import tpu_sc as plsc

BATCH = 4
SEQ = 2048
HIDDEN = 2048
N_ROWS = BATCH * SEQ
NUM_CORES = 2
NUM_SUBCORES = 16
NUM_WORKERS = NUM_CORES * NUM_SUBCORES
ROWS_PER_W = N_ROWS // NUM_WORKERS
CHUNK = 4
NBUF = 4
NUM_CHUNKS = ROWS_PER_W // CHUNK
NUM_STEPS = NUM_CHUNKS // NBUF
LANES = 16
VECS_PER_ROW = HIDDEN // LANES

_mesh = plsc.VectorSubcoreMesh(core_axis_name="c", subcore_axis_name="s")


@functools.partial(
    pl.kernel,
    mesh=_mesh,
    out_type=jax.ShapeDtypeStruct((N_ROWS, HIDDEN), jnp.float32),
    scratch_types=[
        pltpu.VMEM((NUM_CHUNKS, CHUNK), jnp.int32),
        pltpu.VMEM((NUM_CHUNKS, CHUNK), jnp.int32),
        [pltpu.VMEM((CHUNK, HIDDEN), jnp.float32) for _ in range(NBUF)],
        [pltpu.VMEM((CHUNK, HIDDEN), jnp.float32) for _ in range(NBUF)],
        [pltpu.SemaphoreType.DMA for _ in range(NBUF)],
        [pltpu.SemaphoreType.DMA for _ in range(NBUF)],
    ],
)
def _emb_kernel(idw_hbm, idp_hbm, wt_hbm, pt_hbm, out_hbm,
                idw_v, idp_v, wbufs, pbufs, gsems, ssems):
    wid = lax.axis_index("s") * NUM_CORES + lax.axis_index("c")
    base = pl.multiple_of(wid * ROWS_PER_W, ROWS_PER_W)
    pltpu.sync_copy(idw_hbm.at[wid], idw_v)
    pltpu.sync_copy(idp_hbm.at[wid], idp_v)

    def issue_gathers(ci, j):
        pltpu.async_copy(wt_hbm.at[idw_v.at[ci]], wbufs[j], gsems[j])
        pltpu.async_copy(pt_hbm.at[idp_v.at[ci]], pbufs[j], gsems[j])

    def wait_gathers(j):
        pltpu.make_async_copy(wt_hbm.at[idw_v.at[0]], wbufs[j], gsems[j]).wait()
        pltpu.make_async_copy(pt_hbm.at[idp_v.at[0]], pbufs[j], gsems[j]).wait()

    def wait_scatter(j):
        pltpu.make_async_copy(
            wbufs[j], out_hbm.at[pl.ds(base, CHUNK)], ssems[j]).wait()

    for j in range(NBUF):
        issue_gathers(j, j)

    def step(pi, _):
        for j in range(NBUF):
            ci = pi * NBUF + j
            wait_gathers(j)
            wbuf, pbuf = wbufs[j], pbufs[j]

            def vec_body(vi, _):
                col = pl.ds(vi * LANES, LANES)
                for r in range(CHUNK):
                    wbuf[r, col] = wbuf[r, col] + pbuf[r, col]
                return 0

            lax.fori_loop(0, VECS_PER_ROW, vec_body, 0)
            pltpu.async_copy(
                wbuf, out_hbm.at[pl.ds(base + ci * CHUNK, CHUNK)], ssems[j])

            nxt = ci + NBUF

            @pl.when(nxt < NUM_CHUNKS)
            def _():
                wait_scatter(j)
                issue_gathers(nxt, j)

        return 0

    lax.fori_loop(0, NUM_STEPS, step, 0)
    for j in range(NBUF):
        wait_scatter(j)


def kernel(input_ids, position_ids, word_table, pos_table):
    idw = jnp.transpose(input_ids).reshape(NUM_WORKERS, NUM_CHUNKS, CHUNK)
    idp = jnp.transpose(position_ids).reshape(NUM_WORKERS, NUM_CHUNKS, CHUNK)
    out = _emb_kernel(idw.astype(jnp.int32), idp.astype(jnp.int32),
                      word_table, pos_table)
    return out.reshape(SEQ, BATCH, HIDDEN)


# speedup vs baseline: 2.5429x; 1.7726x over previous
import functools

import jax
import jax.numpy as jnp
from jax import lax
from jax.experimental import pallas as pl
from jax.experimental.pallas import tpu as pltpu
from jax.experimental.pallas import tpu_sc as plsc

BATCH = 4
SEQ = 2048
HIDDEN = 2048
N_ROWS = BATCH * SEQ
NUM_CORES = 2
NUM_SUBCORES = 16
NUM_WORKERS = NUM_CORES * NUM_SUBCORES
ROWS_PER_W = N_ROWS // NUM_WORKERS
CHUNK = 4
NBUF = 4
NUM_CHUNKS = ROWS_PER_W // CHUNK
NUM_STEPS = NUM_CHUNKS // NBUF
LANES = 16
VECS_PER_ROW = HIDDEN // LANES

_mesh = plsc.VectorSubcoreMesh(core_axis_name="c", subcore_axis_name="s")


@functools.partial(
    pl.kernel,
    mesh=_mesh,
    out_type=jax.ShapeDtypeStruct((SEQ, BATCH, HIDDEN), jnp.float32),
    scratch_types=[
        pltpu.VMEM((NUM_CHUNKS, CHUNK), jnp.int32),
        pltpu.VMEM((NUM_CHUNKS, CHUNK), jnp.int32),
        [pltpu.VMEM((CHUNK, HIDDEN), jnp.float32) for _ in range(NBUF)],
        [pltpu.VMEM((CHUNK, HIDDEN), jnp.float32) for _ in range(NBUF)],
        [pltpu.SemaphoreType.DMA for _ in range(NBUF)],
        [pltpu.SemaphoreType.DMA for _ in range(NBUF)],
    ],
)
def _emb_kernel(idw_hbm, idp_hbm, wt_hbm, pt_hbm, out_hbm,
                idw_v, idp_v, wbufs, pbufs, gsems, ssems):
    wid = lax.axis_index("s") * NUM_CORES + lax.axis_index("c")
    base = pl.multiple_of(wid * ROWS_PER_W, ROWS_PER_W)
    pltpu.sync_copy(idw_hbm.at[wid], idw_v)
    pltpu.sync_copy(idp_hbm.at[wid], idp_v)

    def issue_gathers(ci, j):
        pltpu.async_copy(wt_hbm.at[idw_v.at[ci]], wbufs[j], gsems[j])
        pltpu.async_copy(pt_hbm.at[idp_v.at[ci]], pbufs[j], gsems[j])

    def wait_gathers(j):
        pltpu.make_async_copy(wt_hbm.at[idw_v.at[0]], wbufs[j], gsems[j]).wait()
        pltpu.make_async_copy(pt_hbm.at[idp_v.at[0]], pbufs[j], gsems[j]).wait()

    def wait_scatter(j):
        pltpu.make_async_copy(wbufs[j], out_hbm.at[base // BATCH], ssems[j]).wait()

    for j in range(NBUF):
        issue_gathers(j, j)

    def step(pi, _):
        for j in range(NBUF):
            ci = pi * NBUF + j
            wait_gathers(j)
            wbuf, pbuf = wbufs[j], pbufs[j]

            def vec_body(vi, _):
                col = pl.ds(vi * LANES, LANES)
                for r in range(CHUNK):
                    wbuf[r, col] = wbuf[r, col] + pbuf[r, col]
                return 0

            lax.fori_loop(0, VECS_PER_ROW, vec_body, 0)
            pltpu.async_copy(wbuf, out_hbm.at[(base + ci * CHUNK) // BATCH], ssems[j])

            nxt = ci + NBUF

            @pl.when(nxt < NUM_CHUNKS)
            def _():
                wait_scatter(j)
                issue_gathers(nxt, j)

        return 0

    lax.fori_loop(0, NUM_STEPS, step, 0)
    for j in range(NBUF):
        wait_scatter(j)


def kernel(input_ids, position_ids, word_table, pos_table):
    idw = jnp.transpose(input_ids).reshape(NUM_WORKERS, NUM_CHUNKS, CHUNK)
    idp = jnp.transpose(position_ids).reshape(NUM_WORKERS, NUM_CHUNKS, CHUNK)
    return _emb_kernel(idw.astype(jnp.int32), idp.astype(jnp.int32),
                       word_table, pos_table)


# speedup vs baseline: 2.5602x; 1.0068x over previous
import functools

import jax
import jax.numpy as jnp
from jax import lax
from jax.experimental import pallas as pl
from jax.experimental.pallas import tpu as pltpu
from jax.experimental.pallas import tpu_sc as plsc

BATCH = 4
SEQ = 2048
HIDDEN = 2048
N_ROWS = BATCH * SEQ
NUM_CORES = 2
NUM_SUBCORES = 16
NUM_WORKERS = NUM_CORES * NUM_SUBCORES
ROWS_PER_W = N_ROWS // NUM_WORKERS
S_PER_W = SEQ // NUM_WORKERS
CHUNK = 4
NBUF = 4
NUM_CHUNKS = ROWS_PER_W // CHUNK
NUM_STEPS = NUM_CHUNKS // NBUF
LANES = 16
VECS_PER_ROW = HIDDEN // LANES

_mesh = plsc.VectorSubcoreMesh(core_axis_name="c", subcore_axis_name="s")


@functools.partial(
    pl.kernel,
    mesh=_mesh,
    out_type=jax.ShapeDtypeStruct((SEQ, BATCH, HIDDEN), jnp.float32),
    scratch_types=[
        pltpu.VMEM((NUM_CHUNKS, CHUNK), jnp.int32),
        pltpu.VMEM((NUM_CHUNKS, CHUNK), jnp.int32),
        [pltpu.VMEM((CHUNK, HIDDEN), jnp.float32) for _ in range(NBUF)],
        [pltpu.VMEM((CHUNK, HIDDEN), jnp.float32) for _ in range(NBUF)],
        [pltpu.SemaphoreType.DMA for _ in range(NBUF)],
        [pltpu.SemaphoreType.DMA for _ in range(NBUF)],
    ],
)
def _emb_kernel(idx_hbm, wt_hbm, pt_hbm, out_hbm,
                idw_v, idp_v, wbufs, pbufs, gsems, ssems):
    wid = lax.axis_index("s") * NUM_CORES + lax.axis_index("c")
    s_base = pl.multiple_of(wid * S_PER_W, S_PER_W)
    pltpu.sync_copy(idx_hbm.at[0, wid], idw_v)
    pltpu.sync_copy(idx_hbm.at[1, wid], idp_v)

    def issue_gathers(ci, j):
        pltpu.async_copy(wt_hbm.at[idw_v.at[ci]], wbufs[j], gsems[j])
        pltpu.async_copy(pt_hbm.at[idp_v.at[ci]], pbufs[j], gsems[j])

    def wait_gathers(j):
        pltpu.make_async_copy(wt_hbm.at[idw_v.at[0]], wbufs[j], gsems[j]).wait()
        pltpu.make_async_copy(pt_hbm.at[idp_v.at[0]], pbufs[j], gsems[j]).wait()

    def wait_scatter(j):
        pltpu.make_async_copy(wbufs[j], out_hbm.at[s_base], ssems[j]).wait()

    for j in range(NBUF):
        issue_gathers(j, j)

    def step(pi, _):
        for j in range(NBUF):
            ci = pi * NBUF + j
            wait_gathers(j)
            wbuf, pbuf = wbufs[j], pbufs[j]

            def vec_body(vi, _):
                col = pl.ds(vi * LANES, LANES)
                for r in range(CHUNK):
                    wbuf[r, col] = wbuf[r, col] + pbuf[r, col]
                return 0

            lax.fori_loop(0, VECS_PER_ROW, vec_body, 0)
            pltpu.async_copy(wbuf, out_hbm.at[s_base + ci], ssems[j])

            nxt = ci + NBUF

            @pl.when(nxt < NUM_CHUNKS)
            def _():
                wait_scatter(j)
                issue_gathers(nxt, j)

        return 0

    lax.fori_loop(0, NUM_STEPS, step, 0)
    for j in range(NBUF):
        wait_scatter(j)


def kernel(input_ids, position_ids, word_table, pos_table):
    ids = jnp.stack([input_ids.astype(jnp.int32), position_ids.astype(jnp.int32)])
    idx = jnp.transpose(ids, (0, 2, 1)).reshape(2, NUM_WORKERS, NUM_CHUNKS, CHUNK)
    return _emb_kernel(idx, word_table, pos_table)
